# baseline (speedup 1.0000x reference)
import jax
import jax.numpy as jnp
from jax.experimental import pallas as pl
from jax.experimental.pallas import tpu as pltpu

B, V, F, D_OBS, D_STATE, HID, K = 256, 128, 64, 256, 512, 1024, 3
BB = 8
G = BB // 2
NBLK = B // BB


def _adj_kernel(obs_ref, state_ref, w1o_ref, w1s_ref, b1_ref, w2_ref, b2_ref,
                sm_ref, adj_ref, ent_ref, hs_ref, lg_ref):
    i = pl.program_id(0)

    @pl.when(i == 0)
    def _precompute():
        hs_ref[...] = (
            jnp.dot(state_ref[...], w1s_ref[...],
                    preferred_element_type=jnp.float32)
            + b1_ref[...]
        )

    @pl.when(i > 0)
    def _vpu_stage():
        rd = (1 - i % 2) * G
        logits = lg_ref[pl.ds(rd, G)]

        m = jnp.max(logits, axis=1, keepdims=True)
        e = jnp.exp(logits - m)
        s = jnp.sum(e, axis=1, keepdims=True)
        sm = e / s
        logp = (logits - m) - jnp.log(s)
        sm_ref[...] = jnp.concatenate([sm[:, :, :F], sm[:, :, F:]], axis=0)

        ent = -jnp.sum(sm * logp, axis=1, keepdims=True)
        ent_lo = jnp.sum(ent[:, :, :F], axis=2, keepdims=True) / F
        ent_hi = jnp.sum(ent[:, :, F:], axis=2, keepdims=True) / F
        ent_ref[...] = jnp.concatenate([ent_lo, ent_hi], axis=0)

        iota = jax.lax.broadcasted_iota(jnp.int32, (G, V, 2 * F), 1)
        v0 = jnp.max(sm, axis=1, keepdims=True)
        i0 = jnp.min(jnp.where(sm == v0, iota, V), axis=1, keepdims=True)
        m0 = iota == i0
        sm1 = jnp.where(m0, -1.0, sm)
        v1 = jnp.max(sm1, axis=1, keepdims=True)
        i1 = jnp.min(jnp.where(sm1 == v1, iota, V), axis=1, keepdims=True)
        m1 = iota == i1
        sm2 = jnp.where(m1, -1.0, sm1)
        v2 = jnp.max(sm2, axis=1, keepdims=True)
        i2 = jnp.min(jnp.where(sm2 == v2, iota, V), axis=1, keepdims=True)
        m2 = iota == i2

        p3 = v0 * v0 * v0
        p2 = 3.0 * v1 * v2 * (v1 + v2)
        p1 = 6.0 * v0 * v1 * v2
        c3 = (p3 > p2) & (p3 > p1)
        c2 = (p2 >= p3) & (p2 > p1)

        cond2 = m0 | (~c3 & m1) | (~(c3 | c2) & m2)
        cond1 = sm > 0.01
        adj = (cond1 & cond2).astype(jnp.int32)
        adj_ref[...] = jnp.concatenate([adj[:, :, :F], adj[:, :, F:]], axis=0)

    @pl.when(i < NBLK)
    def _mxu_stage():
        hs = hs_ref[pl.ds(i * BB, BB), :]
        obs2d = obs_ref[...].reshape(BB * V, D_OBS)
        h0 = jnp.dot(obs2d, w1o_ref[...], preferred_element_type=jnp.float32)
        h = jax.nn.relu(h0.reshape(BB, V, HID) + hs[:, None, :])
        logits = jnp.dot(h.reshape(BB * V, HID), w2_ref[...],
                         preferred_element_type=jnp.float32)
        logits = (logits + b2_ref[...]).reshape(BB, V, F)
        wr = (i % 2) * G
        lg_ref[pl.ds(wr, G)] = jnp.concatenate(
            [logits[:G], logits[G:]], axis=2)


@jax.jit
def kernel(obs, state, W1, b1, W2, b2):
    w1o = W1[:D_OBS]
    w1s = W1[D_OBS:]
    last = NBLK - 1
    grid = (NBLK + 1,)
    sm, adj, ent = pl.pallas_call(
        _adj_kernel,
        grid=grid,
        in_specs=[
            pl.BlockSpec((BB, V, D_OBS),
                         lambda b: (jnp.minimum(b, last), 0, 0)),
            pl.BlockSpec((B, D_STATE), lambda b: (0, 0)),
            pl.BlockSpec((D_OBS, HID), lambda b: (0, 0)),
            pl.BlockSpec((D_STATE, HID), lambda b: (0, 0)),
            pl.BlockSpec((1, HID), lambda b: (0, 0)),
            pl.BlockSpec((HID, F), lambda b: (0, 0)),
            pl.BlockSpec((1, F), lambda b: (0, 0)),
        ],
        out_specs=[
            pl.BlockSpec((BB, V, F), lambda b: (jnp.maximum(b - 1, 0), 0, 0)),
            pl.BlockSpec((BB, V, F), lambda b: (jnp.maximum(b - 1, 0), 0, 0)),
            pl.BlockSpec((BB, 1, 1), lambda b: (jnp.maximum(b - 1, 0), 0, 0)),
        ],
        out_shape=[
            jax.ShapeDtypeStruct((B, V, F), jnp.float32),
            jax.ShapeDtypeStruct((B, V, F), jnp.int32),
            jax.ShapeDtypeStruct((B, 1, 1), jnp.float32),
        ],
        scratch_shapes=[
            pltpu.VMEM((B, HID), jnp.float32),
            pltpu.VMEM((2 * G, V, 2 * F), jnp.float32),
        ],
    )(obs, state, w1o, w1s, b1.reshape(1, HID), W2, b2.reshape(1, F))
    return sm, adj, ent.reshape(B)


# speedup vs baseline: 1.1181x; 1.1181x over previous
import jax
import jax.numpy as jnp
from jax.experimental import pallas as pl
from jax.experimental.pallas import tpu as pltpu

B, V, F, D_OBS, D_STATE, HID, K = 256, 128, 64, 256, 512, 1024, 3
BB = 8
G = BB // 2
NBLK = B // BB


def _adj_kernel(obs_ref, state_ref, w1o_ref, w1s_ref, b1_ref, w2_ref, b2_ref,
                sm_ref, adj_ref, ent_ref, hs_ref, lg_ref):
    i = pl.program_id(0)

    @pl.when(i == 0)
    def _precompute():
        hs_ref[...] = (
            jnp.dot(state_ref[...], w1s_ref[...],
                    preferred_element_type=jnp.float32)
            + b1_ref[...]
        )

    if True:
        rd = (1 - i % 2) * G
        logits = lg_ref[pl.ds(rd, G)]

        m = jnp.max(logits, axis=1, keepdims=True)
        e = jnp.exp(logits - m)
        s = jnp.sum(e, axis=1, keepdims=True)
        sm = e / s
        logp = (logits - m) - jnp.log(s)
        sm_ref[...] = jnp.concatenate([sm[:, :, :F], sm[:, :, F:]], axis=0)

        ent = -jnp.sum(sm * logp, axis=1, keepdims=True)
        ent_lo = jnp.sum(ent[:, :, :F], axis=2, keepdims=True) / F
        ent_hi = jnp.sum(ent[:, :, F:], axis=2, keepdims=True) / F
        ent_ref[...] = jnp.concatenate([ent_lo, ent_hi], axis=0)

        iota = jax.lax.broadcasted_iota(jnp.int32, (G, V, 2 * F), 1)
        v0 = jnp.max(sm, axis=1, keepdims=True)
        i0 = jnp.min(jnp.where(sm == v0, iota, V), axis=1, keepdims=True)
        m0 = iota == i0
        sm1 = jnp.where(m0, -1.0, sm)
        v1 = jnp.max(sm1, axis=1, keepdims=True)
        i1 = jnp.min(jnp.where(sm1 == v1, iota, V), axis=1, keepdims=True)
        m1 = iota == i1
        sm2 = jnp.where(m1, -1.0, sm1)
        v2 = jnp.max(sm2, axis=1, keepdims=True)
        i2 = jnp.min(jnp.where(sm2 == v2, iota, V), axis=1, keepdims=True)
        m2 = iota == i2

        p3 = v0 * v0 * v0
        p2 = 3.0 * v1 * v2 * (v1 + v2)
        p1 = 6.0 * v0 * v1 * v2
        c3 = (p3 > p2) & (p3 > p1)
        c2 = (p2 >= p3) & (p2 > p1)

        cond2 = m0 | (~c3 & m1) | (~(c3 | c2) & m2)
        cond1 = sm > 0.01
        adj = (cond1 & cond2).astype(jnp.int32)
        adj_ref[...] = jnp.concatenate([adj[:, :, :F], adj[:, :, F:]], axis=0)

    if True:
        ic = jnp.minimum(i, NBLK - 1)
        hs = hs_ref[pl.ds(ic * BB, BB), :]
        obs2d = obs_ref[...].reshape(BB * V, D_OBS)
        h0 = jnp.dot(obs2d, w1o_ref[...], preferred_element_type=jnp.float32)
        h = jax.nn.relu(h0.reshape(BB, V, HID) + hs[:, None, :])
        logits = jnp.dot(h.reshape(BB * V, HID), w2_ref[...],
                         preferred_element_type=jnp.float32)
        logits = (logits + b2_ref[...]).reshape(BB, V, F)
        wr = (i % 2) * G
        lg_ref[pl.ds(wr, G)] = jnp.concatenate(
            [logits[:G], logits[G:]], axis=2)


@jax.jit
def kernel(obs, state, W1, b1, W2, b2):
    w1o = W1[:D_OBS]
    w1s = W1[D_OBS:]
    last = NBLK - 1
    grid = (NBLK + 1,)
    sm, adj, ent = pl.pallas_call(
        _adj_kernel,
        grid=grid,
        in_specs=[
            pl.BlockSpec((BB, V, D_OBS),
                         lambda b: (jnp.minimum(b, last), 0, 0)),
            pl.BlockSpec((B, D_STATE), lambda b: (0, 0)),
            pl.BlockSpec((D_OBS, HID), lambda b: (0, 0)),
            pl.BlockSpec((D_STATE, HID), lambda b: (0, 0)),
            pl.BlockSpec((1, HID), lambda b: (0, 0)),
            pl.BlockSpec((HID, F), lambda b: (0, 0)),
            pl.BlockSpec((1, F), lambda b: (0, 0)),
        ],
        out_specs=[
            pl.BlockSpec((BB, V, F), lambda b: (jnp.maximum(b - 1, 0), 0, 0)),
            pl.BlockSpec((BB, V, F), lambda b: (jnp.maximum(b - 1, 0), 0, 0)),
            pl.BlockSpec((BB, 1, 1), lambda b: (jnp.maximum(b - 1, 0), 0, 0)),
        ],
        out_shape=[
            jax.ShapeDtypeStruct((B, V, F), jnp.float32),
            jax.ShapeDtypeStruct((B, V, F), jnp.int32),
            jax.ShapeDtypeStruct((B, 1, 1), jnp.float32),
        ],
        scratch_shapes=[
            pltpu.VMEM((B, HID), jnp.float32),
            pltpu.VMEM((2 * G, V, 2 * F), jnp.float32),
        ],
    )(obs, state, w1o, w1s, b1.reshape(1, HID), W2, b2.reshape(1, F))
    return sm, adj, ent.reshape(B)


# speedup vs baseline: 1.2156x; 1.0872x over previous
import jax
import jax.numpy as jnp
from jax.experimental import pallas as pl
from jax.experimental.pallas import tpu as pltpu

B, V, F, D_OBS, D_STATE, HID, K = 256, 128, 64, 256, 512, 1024, 3
BB = 16
G = BB // 2
NBLK = B // BB


def _adj_kernel(obs_ref, state_ref, w1_ref, b1_ref, w2_ref, b2_ref,
                sm_ref, adj_ref, ent_ref, hs_ref, lg_ref):
    i = pl.program_id(0)

    @pl.when(i == 0)
    def _precompute():
        hs_ref[...] = (
            jnp.dot(state_ref[...], w1_ref[D_OBS:, :],
                    preferred_element_type=jnp.float32)
            + b1_ref[...]
        )

    if True:
        rd = (1 - i % 2) * G
        logits = lg_ref[pl.ds(rd, G)]

        m = jnp.max(logits, axis=1, keepdims=True)
        e = jnp.exp(logits - m)
        s = jnp.sum(e, axis=1, keepdims=True)
        sm = e / s
        logp = (logits - m) - jnp.log(s)
        sm_ref[...] = jnp.concatenate([sm[:, :, :F], sm[:, :, F:]], axis=0)

        ent = -jnp.sum(sm * logp, axis=1, keepdims=True)
        ent_lo = jnp.sum(ent[:, :, :F], axis=2, keepdims=True) / F
        ent_hi = jnp.sum(ent[:, :, F:], axis=2, keepdims=True) / F
        ent_ref[...] = jnp.concatenate([ent_lo, ent_hi], axis=0)

        iota = jax.lax.broadcasted_iota(jnp.int32, (G, V, 2 * F), 1)
        v0 = jnp.max(sm, axis=1, keepdims=True)
        i0 = jnp.min(jnp.where(sm == v0, iota, V), axis=1, keepdims=True)
        m0 = iota == i0
        sm1 = jnp.where(m0, -1.0, sm)
        v1 = jnp.max(sm1, axis=1, keepdims=True)
        i1 = jnp.min(jnp.where(sm1 == v1, iota, V), axis=1, keepdims=True)
        m1 = iota == i1
        sm2 = jnp.where(m1, -1.0, sm1)
        v2 = jnp.max(sm2, axis=1, keepdims=True)
        i2 = jnp.min(jnp.where(sm2 == v2, iota, V), axis=1, keepdims=True)
        m2 = iota == i2

        p3 = v0 * v0 * v0
        p2 = 3.0 * v1 * v2 * (v1 + v2)
        p1 = 6.0 * v0 * v1 * v2
        c3 = (p3 > p2) & (p3 > p1)
        c2 = (p2 >= p3) & (p2 > p1)

        cond2 = m0 | (~c3 & m1) | (~(c3 | c2) & m2)
        cond1 = sm > 0.01
        adj = (cond1 & cond2).astype(jnp.int32)
        adj_ref[...] = jnp.concatenate([adj[:, :, :F], adj[:, :, F:]], axis=0)

    if True:
        ic = jnp.minimum(i, NBLK - 1)
        hs = hs_ref[pl.ds(ic * BB, BB), :]
        obs2d = obs_ref[...].reshape(BB * V, D_OBS)
        h0 = jnp.dot(obs2d, w1_ref[:D_OBS, :], preferred_element_type=jnp.float32)
        h = jax.nn.relu(h0.reshape(BB, V, HID) + hs[:, None, :])
        logits = jnp.dot(h.reshape(BB * V, HID), w2_ref[...],
                         preferred_element_type=jnp.float32)
        logits = (logits + b2_ref[...]).reshape(BB, V, F)
        wr = (i % 2) * G
        lg_ref[pl.ds(wr, G)] = jnp.concatenate(
            [logits[:G], logits[G:]], axis=2)


@jax.jit
def kernel(obs, state, W1, b1, W2, b2):
    last = NBLK - 1
    grid = (NBLK + 1,)
    sm, adj, ent = pl.pallas_call(
        _adj_kernel,
        grid=grid,
        in_specs=[
            pl.BlockSpec((BB, V, D_OBS),
                         lambda b: (jnp.minimum(b, last), 0, 0)),
            pl.BlockSpec((B, D_STATE), lambda b: (0, 0)),
            pl.BlockSpec((D_OBS + D_STATE, HID), lambda b: (0, 0)),
            pl.BlockSpec((1, HID), lambda b: (0, 0)),
            pl.BlockSpec((HID, F), lambda b: (0, 0)),
            pl.BlockSpec((1, F), lambda b: (0, 0)),
        ],
        out_specs=[
            pl.BlockSpec((BB, V, F), lambda b: (jnp.maximum(b - 1, 0), 0, 0)),
            pl.BlockSpec((BB, V, F), lambda b: (jnp.maximum(b - 1, 0), 0, 0)),
            pl.BlockSpec((BB, 1, 1), lambda b: (jnp.maximum(b - 1, 0), 0, 0)),
        ],
        out_shape=[
            jax.ShapeDtypeStruct((B, V, F), jnp.float32),
            jax.ShapeDtypeStruct((B, V, F), jnp.int32),
            jax.ShapeDtypeStruct((B, 1, 1), jnp.float32),
        ],
        scratch_shapes=[
            pltpu.VMEM((B, HID), jnp.float32),
            pltpu.VMEM((2 * G, V, 2 * F), jnp.float32),
        ],
    )(obs, state, W1, b1.reshape(1, HID), W2, b2.reshape(1, F))
    return sm, adj, ent.reshape(B)
